# baseline (speedup 1.0000x reference)
import functools

import jax
import jax.numpy as jnp
from jax import lax
from jax.experimental import pallas as pl
from jax.experimental.pallas import tpu as pltpu
from jax.experimental.pallas import tpu_sc as plsc

B, S, D, E, K = 4, 2048, 768, 8, 2
T = B * S
D2 = D // 2
TR = 512
TM = 512
EPAD = 128
NR = T // TR
TPAD = 2 * T + E * TM
MT = TPAD // TM
NC, NS = 2, 16
NW = NC * NS
TPW = T // NW
CH = 64
WL = 128


def _pack_tc(lo_f32, hi_f32):
    lo = lax.convert_element_type(
        lax.bitcast_convert_type(lo_f32.astype(jnp.bfloat16), jnp.uint16),
        jnp.uint32)
    hi = lax.convert_element_type(
        lax.bitcast_convert_type(hi_f32.astype(jnp.bfloat16), jnp.uint16),
        jnp.uint32)
    return lax.bitcast_convert_type(lo | (hi << 16), jnp.float32)


def _unpack_tc(words_f32):
    w = lax.bitcast_convert_type(words_f32, jnp.uint32)
    lo = lax.bitcast_convert_type(
        lax.convert_element_type(w & 0xFFFF, jnp.uint16), jnp.bfloat16)
    hi = lax.bitcast_convert_type(
        lax.convert_element_type(w >> 16, jnp.uint16), jnp.bfloat16)
    return lo, hi


def _router_kernel(x_ref, wg_ref, bg_ref, meta_ref, offs_ref, te_ref, xp_ref,
                   carry):
    m = pl.program_id(0)
    x = x_ref[...]
    xp_ref[...] = _pack_tc(x[:, :D2], x[:, D2:])
    logits = jnp.dot(x, wg_ref[...], preferred_element_type=jnp.float32)
    logits = logits + bg_ref[0][None, :]
    mx = jnp.max(logits, axis=-1, keepdims=True)
    ex = jnp.exp(logits - mx)
    z = jnp.sum(ex, axis=-1, keepdims=True)
    w0 = 1.0 / z
    ii = lax.broadcasted_iota(jnp.int32, (TR, EPAD), 1)
    e0 = jnp.min(jnp.where(logits == mx, ii, EPAD), axis=-1, keepdims=True)
    sel0 = ii == e0
    l1 = jnp.max(jnp.where(sel0, -jnp.inf, logits), axis=-1, keepdims=True)
    w1 = jnp.exp(l1 - mx) / z
    e1 = jnp.min(jnp.where((logits == l1) & (~sel0), ii, EPAD),
                 axis=-1, keepdims=True)
    sel1 = ii == e1

    mask_f = jnp.where(sel0 | sel1, 1.0, 0.0)
    ti = lax.broadcasted_iota(jnp.int32, (TR, TR), 0)
    tj = lax.broadcasted_iota(jnp.int32, (TR, TR), 1)
    ltri = jnp.where(ti > tj, 1.0, 0.0).astype(jnp.bfloat16)
    rex = jnp.dot(ltri, mask_f.astype(jnp.bfloat16),
                  preferred_element_type=jnp.float32)
    counts = jnp.sum(mask_f, axis=0, keepdims=True)

    @pl.when(m == 0)
    def _():
        carry[...] = jnp.zeros_like(carry)

    rank_g = rex + carry[0:1, :]
    carry[0:1, :] = carry[0:1, :] + counts

    r0 = jnp.sum(jnp.where(sel0, rank_g, 0.0), axis=-1, keepdims=True)
    r1 = jnp.sum(jnp.where(sel1, rank_g, 0.0), axis=-1, keepdims=True)
    meta_ref[...] = (jnp.where(ii == 0, e0.astype(jnp.float32), 0.0)
                     + jnp.where(ii == 1, e1.astype(jnp.float32), 0.0)
                     + jnp.where(ii == 2, r0, 0.0)
                     + jnp.where(ii == 3, r1, 0.0)
                     + jnp.where(ii == 4, w0, 0.0)
                     + jnp.where(ii == 5, w1, 0.0))

    @pl.when(m == NR - 1)
    def _():
        total = carry[0:1, :]
        padded = jnp.floor((total + (TM - 1)) / TM) * TM
        ei = lax.broadcasted_iota(jnp.int32, (EPAD, EPAD), 0)
        ej = lax.broadcasted_iota(jnp.int32, (EPAD, EPAD), 1)
        lt = jnp.where(ei < ej, 1.0, 0.0)
        offs = jnp.dot(padded, lt, preferred_element_type=jnp.float32)
        offs_ref[...] = offs
        ends = offs + padded
        m256 = (lax.broadcasted_iota(jnp.int32, (EPAD, EPAD), 0)
                * TM).astype(jnp.float32)
        live_e = lax.broadcasted_iota(jnp.int32, (EPAD, EPAD), 1) < E
        ge = jnp.where((m256 >= ends) & live_e, 1.0, 0.0)
        te = jnp.sum(ge, axis=-1, keepdims=True)
        te_ref[...] = jnp.broadcast_to(te, (EPAD, EPAD)).astype(jnp.int32)


def _dispatch_body(xp_hbm, meta_hbm, offs_hbm,
                   xs_hbm, ws_hbm, pos0_hbm, pos1_hbm,
                   metab, xb0, xb1, w0a, w1a, w0b, w1b,
                   pos0b, pos1b, offsv, sem):
    wid = lax.axis_index("s") * NC + lax.axis_index("c")
    base = wid * TPW
    pltpu.sync_copy(offs_hbm.at[0, pl.ds(0, 16)], offsv)
    pltpu.sync_copy(meta_hbm.at[pl.ds(base * EPAD, TPW * EPAD)], metab)
    for g in range(TPW // 16):
        rowbase = (lax.iota(jnp.int32, 16) + g * 16) * EPAD
        e0 = plsc.load_gather(metab, [rowbase]).astype(jnp.int32)
        e1 = plsc.load_gather(metab, [rowbase + 1]).astype(jnp.int32)
        r0 = plsc.load_gather(metab, [rowbase + 2])
        r1 = plsc.load_gather(metab, [rowbase + 3])
        p0 = (plsc.load_gather(offsv, [e0]) + r0).astype(jnp.int32)
        p1 = (plsc.load_gather(offsv, [e1]) + r1).astype(jnp.int32)
        pos0b[g // 4, pl.ds((g % 4) * 16, 16)] = p0
        pos1b[g // 4, pl.ds((g % 4) * 16, 16)] = p1
    cpo0 = pltpu.async_copy(pos0b, pos0_hbm.at[wid], sem)
    cpo1 = pltpu.async_copy(pos1b, pos1_hbm.at[wid], sem)

    xbufs = (xb0, xb1)
    wbufs = ((w0a, w1a), (w0b, w1b))
    nch = TPW // CH
    loads = [None] * nch
    loads[0] = pltpu.async_copy(xp_hbm.at[pl.ds(base, CH)], xbufs[0], sem)
    pending = [None, None]
    for ci in range(nch):
        if ci + 1 < nch:
            nslot = (ci + 1) % 2
            if pending[nslot] is not None:
                for h in pending[nslot]:
                    h.wait()
                pending[nslot] = None
            loads[ci + 1] = pltpu.async_copy(
                xp_hbm.at[pl.ds(base + (ci + 1) * CH, CH)],
                xbufs[nslot], sem)
        w0buf, w1buf = wbufs[ci % 2]
        for g in range(CH // 16):
            tok = lax.iota(jnp.int32, 16) + ci * CH + g * 16
            rowbase = tok * EPAD
            w0 = plsc.load_gather(metab, [rowbase + 4])
            w1 = plsc.load_gather(metab, [rowbase + 5])
            wrows = lax.iota(jnp.int32, 16) + g * 16
            wcol0 = jnp.zeros((16,), jnp.int32)
            plsc.store_scatter(w0buf, [wrows, wcol0], w0)
            plsc.store_scatter(w1buf, [wrows, wcol0], w1)
        loads[ci].wait()
        xbuf = xbufs[ci % 2]
        pending[ci % 2] = (
            pltpu.async_copy(xbuf, xs_hbm.at[pos0b.at[ci]], sem),
            pltpu.async_copy(xbuf, xs_hbm.at[pos1b.at[ci]], sem),
            pltpu.async_copy(w0buf, ws_hbm.at[pos0b.at[ci]], sem),
            pltpu.async_copy(w1buf, ws_hbm.at[pos1b.at[ci]], sem),
        )
    for p in pending:
        if p is not None:
            for h in p:
                h.wait()
    cpo0.wait()
    cpo1.wait()


def _mm_kernel(te_ref, xs_ref, we_ref, be_ref, ws_ref, ys_ref):
    m = pl.program_id(0)

    @pl.when(te_ref[m] < E)
    def _():
        e = jnp.minimum(te_ref[m], E - 1)
        lo, hi = _unpack_tc(xs_ref[...])
        xfull = jnp.concatenate([lo, hi], axis=1)
        acc = jnp.dot(xfull, we_ref[e], preferred_element_type=jnp.float32)
        acc = (acc + be_ref[e, 0][None, :]) * ws_ref[:, 0:1]
        ys_ref[...] = _pack_tc(acc[:, :D2], acc[:, D2:])


CHC = 32


def _combine_body(ys_hbm, pos0_hbm, pos1_hbm, out_hbm,
                  y0a, y1a, y0b, y1b, obuf, pos0b, pos1b, sem):
    wid = lax.axis_index("s") * NC + lax.axis_index("c")
    base = wid * TPW
    pltpu.sync_copy(pos0_hbm.at[wid], pos0b)
    pltpu.sync_copy(pos1_hbm.at[wid], pos1b)
    bufs = ((y0a, y1a), (y0b, y1b))
    nch = TPW // CHC

    def issue(c):
        b0, b1 = bufs[c % 2]
        sl = pos0b.at[c // 2, pl.ds((c % 2) * CHC, CHC)]
        sl1 = pos1b.at[c // 2, pl.ds((c % 2) * CHC, CHC)]
        return (pltpu.async_copy(ys_hbm.at[sl], b0, sem),
                pltpu.async_copy(ys_hbm.at[sl1], b1, sem))

    cur = issue(0)
    himask = jnp.full((16,), -65536, jnp.int32)
    for c in range(nch):
        nxt = issue(c + 1) if c + 1 < nch else None
        cur[0].wait()
        cur[1].wait()
        y0buf, y1buf = bufs[c % 2]

        def tok(i, _):
            for j in range(D2 // 16):
                sl = pl.ds(j * 16, 16)
                iv0 = plsc.bitcast(y0buf[i, sl], jnp.int32)
                iv1 = plsc.bitcast(y1buf[i, sl], jnp.int32)
                lo = (plsc.bitcast(iv0 << 16, jnp.float32)
                      + plsc.bitcast(iv1 << 16, jnp.float32))
                hi = (plsc.bitcast(iv0 & himask, jnp.float32)
                      + plsc.bitcast(iv1 & himask, jnp.float32))
                obuf[i, pl.ds(j * 16, 16)] = lo
                obuf[i, pl.ds(D2 + j * 16, 16)] = hi
            return 0

        lax.fori_loop(0, CHC, tok, 0)
        pltpu.sync_copy(obuf, out_hbm.at[pl.ds(base + c * CHC, CHC)])
        cur = nxt


@functools.lru_cache(maxsize=1)
def _sc_kernels():
    mesh = plsc.VectorSubcoreMesh(core_axis_name="c", subcore_axis_name="s")
    params = pltpu.CompilerParams(needs_layout_passes=False)
    dispatch = pl.kernel(
        _dispatch_body, mesh=mesh, compiler_params=params,
        out_type=[
            jax.ShapeDtypeStruct((TPAD, D2), jnp.float32),
            jax.ShapeDtypeStruct((TPAD, WL), jnp.float32),
            jax.ShapeDtypeStruct((NW, TPW // CH, CH), jnp.int32),
            jax.ShapeDtypeStruct((NW, TPW // CH, CH), jnp.int32),
        ],
        scratch_types=[
            pltpu.VMEM((TPW * EPAD,), jnp.float32),
            pltpu.VMEM((CH, D2), jnp.float32),
            pltpu.VMEM((CH, D2), jnp.float32),
            pltpu.VMEM((CH, WL), jnp.float32),
            pltpu.VMEM((CH, WL), jnp.float32),
            pltpu.VMEM((CH, WL), jnp.float32),
            pltpu.VMEM((CH, WL), jnp.float32),
            pltpu.VMEM((TPW // CH, CH), jnp.int32),
            pltpu.VMEM((TPW // CH, CH), jnp.int32),
            pltpu.VMEM((16,), jnp.float32),
            pltpu.SemaphoreType.DMA,
        ])
    combine = pl.kernel(
        _combine_body, mesh=mesh, compiler_params=params,
        out_type=jax.ShapeDtypeStruct((T, D), jnp.float32),
        scratch_types=[
            pltpu.VMEM((CHC, D2), jnp.float32),
            pltpu.VMEM((CHC, D2), jnp.float32),
            pltpu.VMEM((CHC, D2), jnp.float32),
            pltpu.VMEM((CHC, D2), jnp.float32),
            pltpu.VMEM((CHC, D), jnp.float32),
            pltpu.VMEM((TPW // CH, CH), jnp.int32),
            pltpu.VMEM((TPW // CH, CH), jnp.int32),
            pltpu.SemaphoreType.DMA,
        ])
    return dispatch, combine


def kernel(input_tensor, Wg, bg, We, be):
    x = input_tensor.reshape(T, D)
    wg = jnp.pad(Wg, ((0, 0), (0, EPAD - E)))
    bgp = jnp.pad(bg, (0, EPAD - E), constant_values=-1e30).reshape(1, EPAD)
    we_bf = We.astype(jnp.bfloat16)

    meta, offs, te_full, xp = pl.pallas_call(
        _router_kernel,
        grid=(NR,),
        in_specs=[
            pl.BlockSpec((TR, D), lambda m: (m, 0)),
            pl.BlockSpec((D, EPAD), lambda m: (0, 0)),
            pl.BlockSpec((1, EPAD), lambda m: (0, 0)),
        ],
        out_specs=[
            pl.BlockSpec((TR, EPAD), lambda m: (m, 0)),
            pl.BlockSpec((1, EPAD), lambda m: (0, 0)),
            pl.BlockSpec((EPAD, EPAD), lambda m: (0, 0)),
            pl.BlockSpec((TR, D2), lambda m: (m, 0)),
        ],
        out_shape=[
            jax.ShapeDtypeStruct((T, EPAD), jnp.float32),
            jax.ShapeDtypeStruct((1, EPAD), jnp.float32),
            jax.ShapeDtypeStruct((EPAD, EPAD), jnp.int32),
            jax.ShapeDtypeStruct((T, D2), jnp.float32),
        ],
        scratch_shapes=[pltpu.VMEM((8, EPAD), jnp.float32)],
    )(x, wg, bgp)
    te = te_full[:, 0]

    dispatch, combine = _sc_kernels()
    meta_flat = meta.reshape(T * EPAD)
    xs, ws, pos0, pos1 = dispatch(xp, meta_flat, offs)

    ys = pl.pallas_call(
        _mm_kernel,
        grid_spec=pltpu.PrefetchScalarGridSpec(
            num_scalar_prefetch=1,
            grid=(MT,),
            in_specs=[
                pl.BlockSpec((TM, D2), lambda m, te_r: (m, 0)),
                pl.BlockSpec((E, D, D), lambda m, te_r: (0, 0, 0)),
                pl.BlockSpec((E, 1, D), lambda m, te_r: (0, 0, 0)),
                pl.BlockSpec((TM, WL), lambda m, te_r: (m, 0)),
            ],
            out_specs=pl.BlockSpec((TM, D2), lambda m, te_r: (m, 0)),
        ),
        out_shape=jax.ShapeDtypeStruct((TPAD, D2), jnp.float32),
    )(te, xs, we_bf, be.reshape(E, 1, D), ws)

    out = combine(ys, pos0, pos1)
    return out.reshape(B, S, D)


# speedup vs baseline: 1.0130x; 1.0130x over previous
import functools

import jax
import jax.numpy as jnp
from jax import lax
from jax.experimental import pallas as pl
from jax.experimental.pallas import tpu as pltpu
from jax.experimental.pallas import tpu_sc as plsc

B, S, D, E, K = 4, 2048, 768, 8, 2
T = B * S
D2 = D // 2
TR = 1024
TM = 512
EPAD = 128
NR = T // TR
TPAD = 2 * T + E * TM
MT = TPAD // TM
NC, NS = 2, 16
NW = NC * NS
TPW = T // NW
CH = 64
WL = 128


def _pack_tc(lo_f32, hi_f32):
    lo = lax.convert_element_type(
        lax.bitcast_convert_type(lo_f32.astype(jnp.bfloat16), jnp.uint16),
        jnp.uint32)
    hi = lax.convert_element_type(
        lax.bitcast_convert_type(hi_f32.astype(jnp.bfloat16), jnp.uint16),
        jnp.uint32)
    return lax.bitcast_convert_type(lo | (hi << 16), jnp.float32)


def _unpack_tc(words_f32):
    w = lax.bitcast_convert_type(words_f32, jnp.uint32)
    lo = lax.bitcast_convert_type(
        lax.convert_element_type(w & 0xFFFF, jnp.uint16), jnp.bfloat16)
    hi = lax.bitcast_convert_type(
        lax.convert_element_type(w >> 16, jnp.uint16), jnp.bfloat16)
    return lo, hi


def _router_kernel(x_ref, wg_ref, bg_ref, meta_ref, offs_ref, te_ref, xp_ref,
                   carry):
    m = pl.program_id(0)
    x = x_ref[...]
    xp_ref[...] = _pack_tc(x[:, :D2], x[:, D2:])
    logits = jnp.dot(x, wg_ref[...], preferred_element_type=jnp.float32)
    logits = logits + bg_ref[0][None, :]
    mx = jnp.max(logits, axis=-1, keepdims=True)
    ex = jnp.exp(logits - mx)
    z = jnp.sum(ex, axis=-1, keepdims=True)
    w0 = 1.0 / z
    ii = lax.broadcasted_iota(jnp.int32, (TR, EPAD), 1)
    e0 = jnp.min(jnp.where(logits == mx, ii, EPAD), axis=-1, keepdims=True)
    sel0 = ii == e0
    l1 = jnp.max(jnp.where(sel0, -jnp.inf, logits), axis=-1, keepdims=True)
    w1 = jnp.exp(l1 - mx) / z
    e1 = jnp.min(jnp.where((logits == l1) & (~sel0), ii, EPAD),
                 axis=-1, keepdims=True)
    sel1 = ii == e1

    mask_f = jnp.where(sel0 | sel1, 1.0, 0.0)
    ti = lax.broadcasted_iota(jnp.int32, (TR, TR), 0)
    tj = lax.broadcasted_iota(jnp.int32, (TR, TR), 1)
    ltri = jnp.where(ti > tj, 1.0, 0.0).astype(jnp.bfloat16)
    rex = jnp.dot(ltri, mask_f.astype(jnp.bfloat16),
                  preferred_element_type=jnp.float32)
    counts = jnp.sum(mask_f, axis=0, keepdims=True)

    @pl.when(m == 0)
    def _():
        carry[...] = jnp.zeros_like(carry)

    rank_g = rex + carry[0:1, :]
    carry[0:1, :] = carry[0:1, :] + counts

    r0 = jnp.sum(jnp.where(sel0, rank_g, 0.0), axis=-1, keepdims=True)
    r1 = jnp.sum(jnp.where(sel1, rank_g, 0.0), axis=-1, keepdims=True)
    meta_ref[...] = (jnp.where(ii == 0, e0.astype(jnp.float32), 0.0)
                     + jnp.where(ii == 1, e1.astype(jnp.float32), 0.0)
                     + jnp.where(ii == 2, r0, 0.0)
                     + jnp.where(ii == 3, r1, 0.0)
                     + jnp.where(ii == 4, w0, 0.0)
                     + jnp.where(ii == 5, w1, 0.0))

    @pl.when(m == NR - 1)
    def _():
        total = carry[0:1, :]
        padded = jnp.floor((total + (TM - 1)) / TM) * TM
        ei = lax.broadcasted_iota(jnp.int32, (EPAD, EPAD), 0)
        ej = lax.broadcasted_iota(jnp.int32, (EPAD, EPAD), 1)
        lt = jnp.where(ei < ej, 1.0, 0.0)
        offs = jnp.dot(padded, lt, preferred_element_type=jnp.float32)
        offs_ref[...] = offs
        ends = offs + padded
        m256 = (lax.broadcasted_iota(jnp.int32, (EPAD, EPAD), 0)
                * TM).astype(jnp.float32)
        live_e = lax.broadcasted_iota(jnp.int32, (EPAD, EPAD), 1) < E
        ge = jnp.where((m256 >= ends) & live_e, 1.0, 0.0)
        te = jnp.sum(ge, axis=-1, keepdims=True)
        te_ref[...] = jnp.broadcast_to(te, (EPAD, EPAD)).astype(jnp.int32)


def _dispatch_body(xp_hbm, meta_hbm, offs_hbm,
                   xs_hbm, ws_hbm, pos0_hbm, pos1_hbm,
                   metab, xb0, xb1, w0a, w1a, w0b, w1b,
                   pos0b, pos1b, offsv, sem):
    wid = lax.axis_index("s") * NC + lax.axis_index("c")
    base = wid * TPW
    pltpu.sync_copy(offs_hbm.at[0, pl.ds(0, 16)], offsv)
    pltpu.sync_copy(meta_hbm.at[pl.ds(base * EPAD, TPW * EPAD)], metab)
    for g in range(TPW // 16):
        rowbase = (lax.iota(jnp.int32, 16) + g * 16) * EPAD
        e0 = plsc.load_gather(metab, [rowbase]).astype(jnp.int32)
        e1 = plsc.load_gather(metab, [rowbase + 1]).astype(jnp.int32)
        r0 = plsc.load_gather(metab, [rowbase + 2])
        r1 = plsc.load_gather(metab, [rowbase + 3])
        p0 = (plsc.load_gather(offsv, [e0]) + r0).astype(jnp.int32)
        p1 = (plsc.load_gather(offsv, [e1]) + r1).astype(jnp.int32)
        pos0b[g // 4, pl.ds((g % 4) * 16, 16)] = p0
        pos1b[g // 4, pl.ds((g % 4) * 16, 16)] = p1
    cpo0 = pltpu.async_copy(pos0b, pos0_hbm.at[wid], sem)
    cpo1 = pltpu.async_copy(pos1b, pos1_hbm.at[wid], sem)

    xbufs = (xb0, xb1)
    wbufs = ((w0a, w1a), (w0b, w1b))
    nch = TPW // CH
    loads = [None] * nch
    loads[0] = pltpu.async_copy(xp_hbm.at[pl.ds(base, CH)], xbufs[0], sem)
    pending = [None, None]
    for ci in range(nch):
        if ci + 1 < nch:
            nslot = (ci + 1) % 2
            if pending[nslot] is not None:
                for h in pending[nslot]:
                    h.wait()
                pending[nslot] = None
            loads[ci + 1] = pltpu.async_copy(
                xp_hbm.at[pl.ds(base + (ci + 1) * CH, CH)],
                xbufs[nslot], sem)
        w0buf, w1buf = wbufs[ci % 2]
        for g in range(CH // 16):
            tok = lax.iota(jnp.int32, 16) + ci * CH + g * 16
            rowbase = tok * EPAD
            w0 = plsc.load_gather(metab, [rowbase + 4])
            w1 = plsc.load_gather(metab, [rowbase + 5])
            wrows = lax.iota(jnp.int32, 16) + g * 16
            wcol0 = jnp.zeros((16,), jnp.int32)
            plsc.store_scatter(w0buf, [wrows, wcol0], w0)
            plsc.store_scatter(w1buf, [wrows, wcol0], w1)
        loads[ci].wait()
        xbuf = xbufs[ci % 2]
        pending[ci % 2] = (
            pltpu.async_copy(xbuf, xs_hbm.at[pos0b.at[ci]], sem),
            pltpu.async_copy(xbuf, xs_hbm.at[pos1b.at[ci]], sem),
            pltpu.async_copy(w0buf, ws_hbm.at[pos0b.at[ci]], sem),
            pltpu.async_copy(w1buf, ws_hbm.at[pos1b.at[ci]], sem),
        )
    for p in pending:
        if p is not None:
            for h in p:
                h.wait()
    cpo0.wait()
    cpo1.wait()


def _mm_kernel(te_ref, xs_ref, we_ref, be_ref, ws_ref, ys_ref):
    m = pl.program_id(0)

    @pl.when(te_ref[m] < E)
    def _():
        e = jnp.minimum(te_ref[m], E - 1)
        lo, hi = _unpack_tc(xs_ref[...])
        xfull = jnp.concatenate([lo, hi], axis=1)
        acc = jnp.dot(xfull, we_ref[e], preferred_element_type=jnp.float32)
        acc = (acc + be_ref[e, 0][None, :]) * ws_ref[:, 0:1]
        ys_ref[...] = _pack_tc(acc[:, :D2], acc[:, D2:])


CHC = 32


def _combine_body(ys_hbm, pos0_hbm, pos1_hbm, out_hbm,
                  y0a, y1a, y0b, y1b, obuf, pos0b, pos1b, sem):
    wid = lax.axis_index("s") * NC + lax.axis_index("c")
    base = wid * TPW
    pltpu.sync_copy(pos0_hbm.at[wid], pos0b)
    pltpu.sync_copy(pos1_hbm.at[wid], pos1b)
    bufs = ((y0a, y1a), (y0b, y1b))
    nch = TPW // CHC

    def issue(c):
        b0, b1 = bufs[c % 2]
        sl = pos0b.at[c // 2, pl.ds((c % 2) * CHC, CHC)]
        sl1 = pos1b.at[c // 2, pl.ds((c % 2) * CHC, CHC)]
        return (pltpu.async_copy(ys_hbm.at[sl], b0, sem),
                pltpu.async_copy(ys_hbm.at[sl1], b1, sem))

    cur = issue(0)
    himask = jnp.full((16,), -65536, jnp.int32)
    for c in range(nch):
        nxt = issue(c + 1) if c + 1 < nch else None
        cur[0].wait()
        cur[1].wait()
        y0buf, y1buf = bufs[c % 2]

        def tok(i, _):
            for j in range(D2 // 16):
                sl = pl.ds(j * 16, 16)
                iv0 = plsc.bitcast(y0buf[i, sl], jnp.int32)
                iv1 = plsc.bitcast(y1buf[i, sl], jnp.int32)
                lo = (plsc.bitcast(iv0 << 16, jnp.float32)
                      + plsc.bitcast(iv1 << 16, jnp.float32))
                hi = (plsc.bitcast(iv0 & himask, jnp.float32)
                      + plsc.bitcast(iv1 & himask, jnp.float32))
                obuf[i, pl.ds(j * 16, 16)] = lo
                obuf[i, pl.ds(D2 + j * 16, 16)] = hi
            return 0

        lax.fori_loop(0, CHC, tok, 0)
        pltpu.sync_copy(obuf, out_hbm.at[pl.ds(base + c * CHC, CHC)])
        cur = nxt


@functools.lru_cache(maxsize=1)
def _sc_kernels():
    mesh = plsc.VectorSubcoreMesh(core_axis_name="c", subcore_axis_name="s")
    params = pltpu.CompilerParams(needs_layout_passes=False)
    dispatch = pl.kernel(
        _dispatch_body, mesh=mesh, compiler_params=params,
        out_type=[
            jax.ShapeDtypeStruct((TPAD, D2), jnp.float32),
            jax.ShapeDtypeStruct((TPAD, WL), jnp.float32),
            jax.ShapeDtypeStruct((NW, TPW // CH, CH), jnp.int32),
            jax.ShapeDtypeStruct((NW, TPW // CH, CH), jnp.int32),
        ],
        scratch_types=[
            pltpu.VMEM((TPW * EPAD,), jnp.float32),
            pltpu.VMEM((CH, D2), jnp.float32),
            pltpu.VMEM((CH, D2), jnp.float32),
            pltpu.VMEM((CH, WL), jnp.float32),
            pltpu.VMEM((CH, WL), jnp.float32),
            pltpu.VMEM((CH, WL), jnp.float32),
            pltpu.VMEM((CH, WL), jnp.float32),
            pltpu.VMEM((TPW // CH, CH), jnp.int32),
            pltpu.VMEM((TPW // CH, CH), jnp.int32),
            pltpu.VMEM((16,), jnp.float32),
            pltpu.SemaphoreType.DMA,
        ])
    combine = pl.kernel(
        _combine_body, mesh=mesh, compiler_params=params,
        out_type=jax.ShapeDtypeStruct((T, D), jnp.float32),
        scratch_types=[
            pltpu.VMEM((CHC, D2), jnp.float32),
            pltpu.VMEM((CHC, D2), jnp.float32),
            pltpu.VMEM((CHC, D2), jnp.float32),
            pltpu.VMEM((CHC, D2), jnp.float32),
            pltpu.VMEM((CHC, D), jnp.float32),
            pltpu.VMEM((TPW // CH, CH), jnp.int32),
            pltpu.VMEM((TPW // CH, CH), jnp.int32),
            pltpu.SemaphoreType.DMA,
        ])
    return dispatch, combine


def kernel(input_tensor, Wg, bg, We, be):
    x = input_tensor.reshape(T, D)
    wg = jnp.pad(Wg, ((0, 0), (0, EPAD - E)))
    bgp = jnp.pad(bg, (0, EPAD - E), constant_values=-1e30).reshape(1, EPAD)
    we_bf = We.astype(jnp.bfloat16)

    meta, offs, te_full, xp = pl.pallas_call(
        _router_kernel,
        grid=(NR,),
        in_specs=[
            pl.BlockSpec((TR, D), lambda m: (m, 0)),
            pl.BlockSpec((D, EPAD), lambda m: (0, 0)),
            pl.BlockSpec((1, EPAD), lambda m: (0, 0)),
        ],
        out_specs=[
            pl.BlockSpec((TR, EPAD), lambda m: (m, 0)),
            pl.BlockSpec((1, EPAD), lambda m: (0, 0)),
            pl.BlockSpec((EPAD, EPAD), lambda m: (0, 0)),
            pl.BlockSpec((TR, D2), lambda m: (m, 0)),
        ],
        out_shape=[
            jax.ShapeDtypeStruct((T, EPAD), jnp.float32),
            jax.ShapeDtypeStruct((1, EPAD), jnp.float32),
            jax.ShapeDtypeStruct((EPAD, EPAD), jnp.int32),
            jax.ShapeDtypeStruct((T, D2), jnp.float32),
        ],
        scratch_shapes=[pltpu.VMEM((8, EPAD), jnp.float32)],
    )(x, wg, bgp)
    te = te_full[:, 0]

    dispatch, combine = _sc_kernels()
    meta_flat = meta.reshape(T * EPAD)
    xs, ws, pos0, pos1 = dispatch(xp, meta_flat, offs)

    ys = pl.pallas_call(
        _mm_kernel,
        grid_spec=pltpu.PrefetchScalarGridSpec(
            num_scalar_prefetch=1,
            grid=(MT,),
            in_specs=[
                pl.BlockSpec((TM, D2), lambda m, te_r: (m, 0)),
                pl.BlockSpec((E, D, D), lambda m, te_r: (0, 0, 0)),
                pl.BlockSpec((E, 1, D), lambda m, te_r: (0, 0, 0)),
                pl.BlockSpec((TM, WL), lambda m, te_r: (m, 0)),
            ],
            out_specs=pl.BlockSpec((TM, D2), lambda m, te_r: (m, 0)),
        ),
        out_shape=jax.ShapeDtypeStruct((TPAD, D2), jnp.float32),
    )(te, xs, we_bf, be.reshape(E, 1, D), ws)

    out = combine(ys, pos0, pos1)
    return out.reshape(B, S, D)


# speedup vs baseline: 1.0469x; 1.0335x over previous
import functools

import jax
import jax.numpy as jnp
from jax import lax
from jax.experimental import pallas as pl
from jax.experimental.pallas import tpu as pltpu
from jax.experimental.pallas import tpu_sc as plsc

B, S, D, E, K = 4, 2048, 768, 8, 2
T = B * S
D2 = D // 2
TR = 1024
TM = 512
EPAD = 128
NR = T // TR
TPAD = 2 * T + E * TM
MT = TPAD // TM
NC, NS = 2, 16
NW = NC * NS
TPW = T // NW
CH = 64
WL = 128


def _pack_tc(lo_f32, hi_f32):
    lo = lax.convert_element_type(
        lax.bitcast_convert_type(lo_f32.astype(jnp.bfloat16), jnp.uint16),
        jnp.uint32)
    hi = lax.convert_element_type(
        lax.bitcast_convert_type(hi_f32.astype(jnp.bfloat16), jnp.uint16),
        jnp.uint32)
    return lax.bitcast_convert_type(lo | (hi << 16), jnp.float32)


def _unpack_tc(words_f32):
    w = lax.bitcast_convert_type(words_f32, jnp.uint32)
    lo = lax.bitcast_convert_type(
        lax.convert_element_type(w & 0xFFFF, jnp.uint16), jnp.bfloat16)
    hi = lax.bitcast_convert_type(
        lax.convert_element_type(w >> 16, jnp.uint16), jnp.bfloat16)
    return lo, hi


def _router_kernel(x_ref, wg_ref, bg_ref, meta_ref, offs_ref, te_ref, xp_ref,
                   carry):
    m = pl.program_id(0)
    x = x_ref[...]
    xp_ref[...] = _pack_tc(x[:, :D2], x[:, D2:])
    logits = jnp.dot(x, wg_ref[...], preferred_element_type=jnp.float32)
    logits = logits + bg_ref[0][None, :]
    mx = jnp.max(logits, axis=-1, keepdims=True)
    ex = jnp.exp(logits - mx)
    z = jnp.sum(ex, axis=-1, keepdims=True)
    w0 = 1.0 / z
    ii = lax.broadcasted_iota(jnp.int32, (TR, EPAD), 1)
    e0 = jnp.min(jnp.where(logits == mx, ii, EPAD), axis=-1, keepdims=True)
    sel0 = ii == e0
    l1 = jnp.max(jnp.where(sel0, -jnp.inf, logits), axis=-1, keepdims=True)
    w1 = jnp.exp(l1 - mx) / z
    e1 = jnp.min(jnp.where((logits == l1) & (~sel0), ii, EPAD),
                 axis=-1, keepdims=True)
    sel1 = ii == e1

    mask_f = jnp.where(sel0 | sel1, 1.0, 0.0)
    ti = lax.broadcasted_iota(jnp.int32, (TR, TR), 0)
    tj = lax.broadcasted_iota(jnp.int32, (TR, TR), 1)
    ltri = jnp.where(ti > tj, 1.0, 0.0).astype(jnp.bfloat16)
    rex = jnp.dot(ltri, mask_f.astype(jnp.bfloat16),
                  preferred_element_type=jnp.float32)
    counts = jnp.sum(mask_f, axis=0, keepdims=True)

    @pl.when(m == 0)
    def _():
        carry[...] = jnp.zeros_like(carry)

    rank_g = rex + carry[0:1, :]
    carry[0:1, :] = carry[0:1, :] + counts

    r0 = jnp.sum(jnp.where(sel0, rank_g, 0.0), axis=-1, keepdims=True)
    r1 = jnp.sum(jnp.where(sel1, rank_g, 0.0), axis=-1, keepdims=True)
    meta_ref[...] = (jnp.where(ii == 0, e0.astype(jnp.float32), 0.0)
                     + jnp.where(ii == 1, e1.astype(jnp.float32), 0.0)
                     + jnp.where(ii == 2, r0, 0.0)
                     + jnp.where(ii == 3, r1, 0.0)
                     + jnp.where(ii == 4, w0, 0.0)
                     + jnp.where(ii == 5, w1, 0.0))

    @pl.when(m == NR - 1)
    def _():
        total = carry[0:1, :]
        padded = jnp.floor((total + (TM - 1)) / TM) * TM
        ei = lax.broadcasted_iota(jnp.int32, (EPAD, EPAD), 0)
        ej = lax.broadcasted_iota(jnp.int32, (EPAD, EPAD), 1)
        lt = jnp.where(ei < ej, 1.0, 0.0)
        offs = jnp.dot(padded, lt, preferred_element_type=jnp.float32)
        offs_ref[...] = offs
        ends = offs + padded
        m256 = (lax.broadcasted_iota(jnp.int32, (EPAD, EPAD), 0)
                * TM).astype(jnp.float32)
        live_e = lax.broadcasted_iota(jnp.int32, (EPAD, EPAD), 1) < E
        ge = jnp.where((m256 >= ends) & live_e, 1.0, 0.0)
        te = jnp.sum(ge, axis=-1, keepdims=True)
        te_ref[...] = jnp.broadcast_to(te, (EPAD, EPAD)).astype(jnp.int32)


def _dispatch_body(xp_hbm, meta_hbm, offs_hbm,
                   xs_hbm, ws_hbm, pos0_hbm, pos1_hbm,
                   metab, xb0, xb1, w0a, w1a, w0b, w1b,
                   pos0b, pos1b, offsv, sem):
    wid = lax.axis_index("s") * NC + lax.axis_index("c")
    base = wid * TPW
    pltpu.sync_copy(offs_hbm.at[0, pl.ds(0, 16)], offsv)
    pltpu.sync_copy(meta_hbm.at[pl.ds(base * EPAD, TPW * EPAD)], metab)
    for g in range(TPW // 16):
        rowbase = (lax.iota(jnp.int32, 16) + g * 16) * EPAD
        e0 = plsc.load_gather(metab, [rowbase]).astype(jnp.int32)
        e1 = plsc.load_gather(metab, [rowbase + 1]).astype(jnp.int32)
        r0 = plsc.load_gather(metab, [rowbase + 2])
        r1 = plsc.load_gather(metab, [rowbase + 3])
        p0 = (plsc.load_gather(offsv, [e0]) + r0).astype(jnp.int32)
        p1 = (plsc.load_gather(offsv, [e1]) + r1).astype(jnp.int32)
        pos0b[g // 4, pl.ds((g % 4) * 16, 16)] = p0
        pos1b[g // 4, pl.ds((g % 4) * 16, 16)] = p1
    cpo0 = pltpu.async_copy(pos0b, pos0_hbm.at[wid], sem)
    cpo1 = pltpu.async_copy(pos1b, pos1_hbm.at[wid], sem)

    xbufs = (xb0, xb1)
    wbufs = ((w0a, w1a), (w0b, w1b))
    nch = TPW // CH
    loads = [None] * nch
    loads[0] = pltpu.async_copy(xp_hbm.at[pl.ds(base, CH)], xbufs[0], sem)
    pending = [None, None]
    for ci in range(nch):
        if ci + 1 < nch:
            nslot = (ci + 1) % 2
            if pending[nslot] is not None:
                for h in pending[nslot]:
                    h.wait()
                pending[nslot] = None
            loads[ci + 1] = pltpu.async_copy(
                xp_hbm.at[pl.ds(base + (ci + 1) * CH, CH)],
                xbufs[nslot], sem)
        w0buf, w1buf = wbufs[ci % 2]
        for g in range(CH // 16):
            tok = lax.iota(jnp.int32, 16) + ci * CH + g * 16
            rowbase = tok * EPAD
            w0 = plsc.load_gather(metab, [rowbase + 4])
            w1 = plsc.load_gather(metab, [rowbase + 5])
            wrows = lax.iota(jnp.int32, 16) + g * 16
            wcol0 = jnp.zeros((16,), jnp.int32)
            plsc.store_scatter(w0buf, [wrows, wcol0], w0)
            plsc.store_scatter(w1buf, [wrows, wcol0], w1)
        loads[ci].wait()
        xbuf = xbufs[ci % 2]
        pending[ci % 2] = (
            pltpu.async_copy(xbuf, xs_hbm.at[pos0b.at[ci]], sem),
            pltpu.async_copy(xbuf, xs_hbm.at[pos1b.at[ci]], sem),
            pltpu.async_copy(w0buf, ws_hbm.at[pos0b.at[ci]], sem),
            pltpu.async_copy(w1buf, ws_hbm.at[pos1b.at[ci]], sem),
        )
    for p in pending:
        if p is not None:
            for h in p:
                h.wait()
    cpo0.wait()
    cpo1.wait()


def _mm_kernel(te_ref, xs_ref, we_ref, be_ref, ws_ref, ys_ref):
    m = pl.program_id(0)

    @pl.when(te_ref[m] < E)
    def _():
        e = jnp.minimum(te_ref[m], E - 1)
        lo, hi = _unpack_tc(xs_ref[...])
        xfull = jnp.concatenate([lo, hi], axis=1)
        acc = jnp.dot(xfull, we_ref[e], preferred_element_type=jnp.float32)
        acc = (acc + be_ref[e, 0][None, :]) * ws_ref[:, 0:1]
        ys_ref[...] = _pack_tc(acc[:, :D2], acc[:, D2:])


CHC = 32


def _combine_body(ys_hbm, pos0_hbm, pos1_hbm, out_hbm,
                  y0a, y1a, y0b, y1b, oa, ob, pos0b, pos1b, sem):
    wid = lax.axis_index("s") * NC + lax.axis_index("c")
    base = wid * TPW
    pltpu.sync_copy(pos0_hbm.at[wid], pos0b)
    pltpu.sync_copy(pos1_hbm.at[wid], pos1b)
    bufs = ((y0a, y1a), (y0b, y1b))
    nch = TPW // CHC

    def issue(c):
        b0, b1 = bufs[c % 2]
        sl = pos0b.at[c // 2, pl.ds((c % 2) * CHC, CHC)]
        sl1 = pos1b.at[c // 2, pl.ds((c % 2) * CHC, CHC)]
        return (pltpu.async_copy(ys_hbm.at[sl], b0, sem),
                pltpu.async_copy(ys_hbm.at[sl1], b1, sem))

    cur = issue(0)
    himask = jnp.full((16,), -65536, jnp.int32)
    obufs = (oa, ob)
    outp = [None, None]
    for c in range(nch):
        nxt = issue(c + 1) if c + 1 < nch else None
        if outp[c % 2] is not None:
            outp[c % 2].wait()
            outp[c % 2] = None
        obuf = obufs[c % 2]
        cur[0].wait()
        cur[1].wait()
        y0buf, y1buf = bufs[c % 2]

        def tok(i, _):
            for j in range(D2 // 16):
                sl = pl.ds(j * 16, 16)
                iv0 = plsc.bitcast(y0buf[i, sl], jnp.int32)
                iv1 = plsc.bitcast(y1buf[i, sl], jnp.int32)
                lo = (plsc.bitcast(iv0 << 16, jnp.float32)
                      + plsc.bitcast(iv1 << 16, jnp.float32))
                hi = (plsc.bitcast(iv0 & himask, jnp.float32)
                      + plsc.bitcast(iv1 & himask, jnp.float32))
                obuf[i, pl.ds(j * 16, 16)] = lo
                obuf[i, pl.ds(D2 + j * 16, 16)] = hi
            return 0

        lax.fori_loop(0, CHC, tok, 0)
        outp[c % 2] = pltpu.async_copy(
            obuf, out_hbm.at[pl.ds(base + c * CHC, CHC)], sem)
        cur = nxt
    for h in outp:
        if h is not None:
            h.wait()


@functools.lru_cache(maxsize=1)
def _sc_kernels():
    mesh = plsc.VectorSubcoreMesh(core_axis_name="c", subcore_axis_name="s")
    params = pltpu.CompilerParams(needs_layout_passes=False)
    dispatch = pl.kernel(
        _dispatch_body, mesh=mesh, compiler_params=params,
        out_type=[
            jax.ShapeDtypeStruct((TPAD, D2), jnp.float32),
            jax.ShapeDtypeStruct((TPAD, WL), jnp.float32),
            jax.ShapeDtypeStruct((NW, TPW // CH, CH), jnp.int32),
            jax.ShapeDtypeStruct((NW, TPW // CH, CH), jnp.int32),
        ],
        scratch_types=[
            pltpu.VMEM((TPW * EPAD,), jnp.float32),
            pltpu.VMEM((CH, D2), jnp.float32),
            pltpu.VMEM((CH, D2), jnp.float32),
            pltpu.VMEM((CH, WL), jnp.float32),
            pltpu.VMEM((CH, WL), jnp.float32),
            pltpu.VMEM((CH, WL), jnp.float32),
            pltpu.VMEM((CH, WL), jnp.float32),
            pltpu.VMEM((TPW // CH, CH), jnp.int32),
            pltpu.VMEM((TPW // CH, CH), jnp.int32),
            pltpu.VMEM((16,), jnp.float32),
            pltpu.SemaphoreType.DMA,
        ])
    combine = pl.kernel(
        _combine_body, mesh=mesh, compiler_params=params,
        out_type=jax.ShapeDtypeStruct((T, D), jnp.float32),
        scratch_types=[
            pltpu.VMEM((CHC, D2), jnp.float32),
            pltpu.VMEM((CHC, D2), jnp.float32),
            pltpu.VMEM((CHC, D2), jnp.float32),
            pltpu.VMEM((CHC, D2), jnp.float32),
            pltpu.VMEM((CHC, D), jnp.float32),
            pltpu.VMEM((CHC, D), jnp.float32),
            pltpu.VMEM((TPW // CH, CH), jnp.int32),
            pltpu.VMEM((TPW // CH, CH), jnp.int32),
            pltpu.SemaphoreType.DMA,
        ])
    return dispatch, combine


def kernel(input_tensor, Wg, bg, We, be):
    x = input_tensor.reshape(T, D)
    wg = jnp.pad(Wg, ((0, 0), (0, EPAD - E)))
    bgp = jnp.pad(bg, (0, EPAD - E), constant_values=-1e30).reshape(1, EPAD)
    we_bf = We.astype(jnp.bfloat16)

    meta, offs, te_full, xp = pl.pallas_call(
        _router_kernel,
        grid=(NR,),
        in_specs=[
            pl.BlockSpec((TR, D), lambda m: (m, 0)),
            pl.BlockSpec((D, EPAD), lambda m: (0, 0)),
            pl.BlockSpec((1, EPAD), lambda m: (0, 0)),
        ],
        out_specs=[
            pl.BlockSpec((TR, EPAD), lambda m: (m, 0)),
            pl.BlockSpec((1, EPAD), lambda m: (0, 0)),
            pl.BlockSpec((EPAD, EPAD), lambda m: (0, 0)),
            pl.BlockSpec((TR, D2), lambda m: (m, 0)),
        ],
        out_shape=[
            jax.ShapeDtypeStruct((T, EPAD), jnp.float32),
            jax.ShapeDtypeStruct((1, EPAD), jnp.float32),
            jax.ShapeDtypeStruct((EPAD, EPAD), jnp.int32),
            jax.ShapeDtypeStruct((T, D2), jnp.float32),
        ],
        scratch_shapes=[pltpu.VMEM((8, EPAD), jnp.float32)],
    )(x, wg, bgp)
    te = te_full[:, 0]

    dispatch, combine = _sc_kernels()
    meta_flat = meta.reshape(T * EPAD)
    xs, ws, pos0, pos1 = dispatch(xp, meta_flat, offs)

    ys = pl.pallas_call(
        _mm_kernel,
        grid_spec=pltpu.PrefetchScalarGridSpec(
            num_scalar_prefetch=1,
            grid=(MT,),
            in_specs=[
                pl.BlockSpec((TM, D2), lambda m, te_r: (m, 0)),
                pl.BlockSpec((E, D, D), lambda m, te_r: (0, 0, 0)),
                pl.BlockSpec((E, 1, D), lambda m, te_r: (0, 0, 0)),
                pl.BlockSpec((TM, WL), lambda m, te_r: (m, 0)),
            ],
            out_specs=pl.BlockSpec((TM, D2), lambda m, te_r: (m, 0)),
        ),
        out_shape=jax.ShapeDtypeStruct((TPAD, D2), jnp.float32),
    )(te, xs, we_bf, be.reshape(E, 1, D), ws)

    out = combine(ys, pos0, pos1)
    return out.reshape(B, S, D)
